# baseline (speedup 1.0000x reference)
import jax
import jax.numpy as jnp
from jax.experimental import pallas as pl


def _block_rows(n):
    for b in (400, 200, 100, 80, 40, 16, 8):
        if n % b == 0:
            return b
    return n


def _k1_body(adj_ref, x_ref, w1_ref, d_ref, m1_ref):
    a = adj_ref[...]
    s = jnp.sum(a, axis=1) + 1.0
    d = jnp.where(s > 0, jax.lax.rsqrt(s), 0.0)
    d_ref[...] = d[:, None]
    y = jnp.dot(x_ref[...], w1_ref[...], preferred_element_type=jnp.float32)
    m1_ref[...] = d[:, None] * y


def _k2_body(adj_ref, m1f_ref, m1b_ref, d_ref, w2_ref, m2_ref):
    t = jnp.dot(adj_ref[...], m1f_ref[...],
                preferred_element_type=jnp.float32) + m1b_ref[...]
    d = d_ref[...]
    m2_ref[...] = (d * d) * jnp.dot(t, w2_ref[...],
                                    preferred_element_type=jnp.float32)


def _k3_body(adj_ref, m2f_ref, m2b_ref, d_ref, out_ref):
    pre = d_ref[...] * (jnp.dot(adj_ref[...], m2f_ref[...],
                                preferred_element_type=jnp.float32)
                        + m2b_ref[...])
    m = jnp.max(pre, axis=1, keepdims=True)
    e = pre - m
    lse = jnp.log(jnp.sum(jnp.exp(e), axis=1, keepdims=True))
    out_ref[...] = e - lse


def kernel(x, adj, W1, W2):
    n, f = x.shape
    h = W1.shape[1]
    c = W2.shape[1]
    br = _block_rows(n)
    grid = (n // br,)

    def row_blk(r, cdim):
        return pl.BlockSpec((r, cdim), lambda i: (i, 0))

    def full(shape):
        return pl.BlockSpec(shape, lambda i: (0, 0))

    d, m1 = pl.pallas_call(
        _k1_body,
        grid=grid,
        in_specs=[row_blk(br, n), row_blk(br, f), full((f, h))],
        out_specs=[row_blk(br, 1), row_blk(br, h)],
        out_shape=[jax.ShapeDtypeStruct((n, 1), jnp.float32),
                   jax.ShapeDtypeStruct((n, h), jnp.float32)],
    )(adj, x, W1)

    m2 = pl.pallas_call(
        _k2_body,
        grid=grid,
        in_specs=[row_blk(br, n), full((n, h)), row_blk(br, h),
                  row_blk(br, 1), full((h, c))],
        out_specs=row_blk(br, c),
        out_shape=jax.ShapeDtypeStruct((n, c), jnp.float32),
    )(adj, m1, m1, d, W2)

    out = pl.pallas_call(
        _k3_body,
        grid=grid,
        in_specs=[row_blk(br, n), full((n, c)), row_blk(br, c),
                  row_blk(br, 1)],
        out_specs=row_blk(br, c),
        out_shape=jax.ShapeDtypeStruct((n, c), jnp.float32),
    )(adj, m2, m2, d)
    return out
